# baseline (speedup 1.0000x reference)
import functools

import jax
import jax.numpy as jnp
from jax import lax
from jax.experimental import pallas as pl
from jax.experimental.pallas import tpu as pltpu
from jax.experimental.pallas import tpu_sc as plsc

HIDDEN = 1024
NUM_CORES = 2
NUM_SUBCORES = 16
NW = NUM_CORES * NUM_SUBCORES
CHUNK = 64


@functools.partial(jax.jit, static_argnames=())
def _embed(table, idx_flat):
    n = idx_flat.shape[0]
    b_per_w = n // NW
    n_chunks = b_per_w // CHUNK

    mesh = plsc.VectorSubcoreMesh(core_axis_name="c", subcore_axis_name="s")

    @functools.partial(
        pl.kernel,
        out_type=jax.ShapeDtypeStruct((n, HIDDEN), jnp.float32),
        mesh=mesh,
        scratch_types=[
            pltpu.VMEM((b_per_w,), jnp.int32),
            pltpu.VMEM((CHUNK, HIDDEN), jnp.float32),
            pltpu.SemaphoreType.DMA,
        ],
    )
    def k(table_hbm, idx_hbm, out_hbm, idx_v, rows_v, gsem):
        wid = lax.axis_index("s") * NUM_CORES + lax.axis_index("c")
        base = wid * b_per_w
        pltpu.sync_copy(idx_hbm.at[pl.ds(base, b_per_w)], idx_v)

        def chunk_body(i, carry):
            row0 = i * CHUNK
            pltpu.async_copy(
                table_hbm.at[idx_v.at[pl.ds(row0, CHUNK)]], rows_v, gsem
            ).wait()
            pltpu.sync_copy(rows_v, out_hbm.at[pl.ds(base + row0, CHUNK)])
            return carry

        lax.fori_loop(0, n_chunks, chunk_body, 0)

    return k(table, idx_flat)


def kernel(segments, table):
    b, s = segments.shape
    out = _embed(table, segments.reshape(b * s))
    return out.reshape(b, s, HIDDEN)


# speedup vs baseline: 13.3239x; 13.3239x over previous
import functools

import jax
import jax.numpy as jnp
from jax import lax
from jax.experimental import pallas as pl
from jax.experimental.pallas import tpu as pltpu
from jax.experimental.pallas import tpu_sc as plsc

HIDDEN = 1024
LANES = 16
NUM_CORES = 2
NUM_SUBCORES = 16
NW = NUM_CORES * NUM_SUBCORES


def _embed(table, idx_flat):
    n = idx_flat.shape[0]
    b_per_w = n // NW
    n_groups = b_per_w // LANES

    mesh = plsc.VectorSubcoreMesh(core_axis_name="c", subcore_axis_name="s")

    @functools.partial(
        pl.kernel,
        out_type=jax.ShapeDtypeStruct((n, HIDDEN), jnp.float32),
        mesh=mesh,
        scratch_types=[
            pltpu.VMEM((b_per_w,), jnp.int32),
            pltpu.VMEM((2, HIDDEN), jnp.float32),
            pltpu.SemaphoreType.DMA,
        ],
    )
    def k(table_hbm, idx_hbm, out_hbm, idx_v, tab_v, sem):
        wid = lax.axis_index("s") * NUM_CORES + lax.axis_index("c")
        base = wid * b_per_w
        pltpu.sync_copy(idx_hbm.at[pl.ds(base, b_per_w)], idx_v)
        pltpu.sync_copy(table_hbm, tab_v)

        def group_body(g, carry):
            segv = idx_v[pl.ds(g * LANES, LANES)]
            for rr in range(LANES):
                pltpu.make_async_copy(
                    tab_v.at[segv[rr]],
                    out_hbm.at[base + g * LANES + rr],
                    sem,
                ).start()

            @pl.when(g > 0)
            def _drain_prev():
                for _ in range(LANES):
                    pltpu.make_async_copy(
                        tab_v.at[0], out_hbm.at[base], sem
                    ).wait()

            return carry

        lax.fori_loop(0, n_groups, group_body, 0)
        for _ in range(LANES):
            pltpu.make_async_copy(tab_v.at[0], out_hbm.at[base], sem).wait()

    return k(table, idx_flat)


def kernel(segments, table):
    b, s = segments.shape
    out = _embed(table, segments.reshape(b * s))
    return out.reshape(b, s, HIDDEN)
